# baseline (speedup 1.0000x reference)
import functools

import jax
import jax.numpy as jnp
import numpy as np
from jax import lax
from jax.experimental import pallas as pl
from jax.experimental.pallas import tpu as pltpu
from jax.experimental.pallas import tpu_sc as plsc

_NBUF = 4
_LANE = 128


def _sin_pe(max_len, d):
    pos = np.arange(max_len, dtype=np.float32)[:, None]
    div = np.exp(np.arange(0, d, 2, dtype=np.float32) * (-np.log(10000.0) / d))
    pe = np.zeros((max_len, d), dtype=np.float32)
    pe[:, 0::2] = np.sin(pos * div)
    pe[:, 1::2] = np.cos(pos * div)
    return pe


@functools.lru_cache(maxsize=None)
def _build(B, L, D, V):
    info = plsc.get_sparse_core_info()
    NC, NS = info.num_cores, info.num_subcores
    NW = NC * NS
    assert B % NW == 0
    n_chunks = B // NW
    assert n_chunks % _NBUF == 0 and n_chunks >= _NBUF
    G0 = 128
    G1 = L - G0
    mesh = plsc.VectorSubcoreMesh(core_axis_name="c", subcore_axis_name="s")

    @functools.partial(
        pl.kernel,
        out_type=jax.ShapeDtypeStruct((B, L, _LANE), jnp.float32),
        mesh=mesh,
        scratch_types=[pltpu.VMEM_SHARED((L, _LANE), jnp.float32)]
        + [pltpu.VMEM((L,), jnp.int32) for _ in range(_NBUF)]
        + [pltpu.VMEM((L, _LANE), jnp.float32) for _ in range(_NBUF)]
        + [pltpu.SemaphoreType.DMA for _ in range(4 * _NBUF)],
        compiler_params=pltpu.CompilerParams(use_tc_tiling_on_sc=True),
    )
    def emb(x_hbm, pe_hbm, table_hbm, out_hbm, pe_v, *refs):
        idxs = refs[:_NBUF]
        bufs = refs[_NBUF : 2 * _NBUF]
        sem_i = refs[2 * _NBUF : 3 * _NBUF]
        sem_g = refs[3 * _NBUF : 4 * _NBUF]
        sem_o = refs[4 * _NBUF : 5 * _NBUF]
        sem_p = refs[5 * _NBUF :]

        wid = lax.axis_index("s") * NC + lax.axis_index("c")
        seq0 = wid * n_chunks

        @pl.when(lax.axis_index("s") == 0)
        def _():
            pltpu.sync_copy(pe_hbm, pe_v)

        plsc.subcore_barrier()

        def start_idx(c, b):
            pltpu.async_copy(x_hbm.at[seq0 + c], idxs[b], sem_i[b])

        def wait_idx(b):
            pltpu.make_async_copy(x_hbm.at[0], idxs[b], sem_i[b]).wait()

        def start_prefill(b):
            pltpu.async_copy(pe_v, bufs[b], sem_p[b])

        def wait_prefill(b):
            pltpu.make_async_copy(pe_v, bufs[b], sem_p[b]).wait()

        def start_gather(b):
            pltpu.async_copy(
                table_hbm.at[idxs[b].at[pl.ds(0, G0)]],
                bufs[b].at[pl.ds(0, G0)],
                sem_g[b],
                add=True,
            )
            pltpu.async_copy(
                table_hbm.at[idxs[b].at[pl.ds(G0, G1)]],
                bufs[b].at[pl.ds(G0, G1)],
                sem_g[b],
                add=True,
            )

        def wait_gather(b):
            pltpu.make_async_copy(
                table_hbm.at[idxs[b].at[pl.ds(0, G0)]], bufs[b].at[pl.ds(0, G0)], sem_g[b]
            ).wait()
            pltpu.make_async_copy(
                table_hbm.at[idxs[b].at[pl.ds(0, G1)]], bufs[b].at[pl.ds(G0, G1)], sem_g[b]
            ).wait()

        def start_out(c, b):
            pltpu.async_copy(bufs[b], out_hbm.at[seq0 + c], sem_o[b])

        def wait_out(b):
            pltpu.make_async_copy(bufs[b], out_hbm.at[0], sem_o[b]).wait()

        for b in range(3):
            start_idx(b, b)
            start_prefill(b)
        wait_idx(0)
        wait_prefill(0)
        start_gather(0)

        def grp_body(grp, carry):
            c_base = grp * _NBUF
            for bb in range(_NBUF):
                c = c_base + bb
                wait_gather(bb)
                start_out(c, bb)

                c3 = c + 3
                b3 = (bb + 3) % _NBUF

                @pl.when(jnp.logical_and(c3 < n_chunks, c3 >= _NBUF))
                def _():
                    wait_out(b3)

                @pl.when(c3 < n_chunks)
                def _():
                    start_idx(c3, b3)
                    start_prefill(b3)

                c1 = c + 1
                b1 = (bb + 1) % _NBUF

                @pl.when(c1 < n_chunks)
                def _():
                    wait_idx(b1)
                    wait_prefill(b1)
                    start_gather(b1)

            return carry

        lax.fori_loop(0, n_chunks // _NBUF, grp_body, 0)
        for b in range(_NBUF):
            wait_out(b)

    return emb


def kernel(x, table):
    B, L = x.shape
    V, D = table.shape
    pe = np.zeros((L, _LANE), dtype=np.float32)
    pe[:, :D] = _sin_pe(L, D)
    exp_mat = np.zeros((D, _LANE), dtype=np.float32)
    exp_mat[np.arange(D), np.arange(D)] = 1.0
    table_wide = jax.lax.dot_general(
        table, jnp.asarray(exp_mat), (((1,), (0,)), ((), ())),
        precision=jax.lax.Precision.HIGHEST,
    )
    out_wide = _build(B, L, D, V)(x.astype(jnp.int32), jnp.asarray(pe), table_wide)
    return out_wide[:, :, :D]


# speedup vs baseline: 1.2943x; 1.2943x over previous
import functools

import jax
import jax.numpy as jnp
import numpy as np
from jax import lax
from jax.experimental import pallas as pl
from jax.experimental.pallas import tpu as pltpu
from jax.experimental.pallas import tpu_sc as plsc

_NBUF = 4
_LANE = 128


def _sin_pe(max_len, d):
    pos = np.arange(max_len, dtype=np.float32)[:, None]
    div = np.exp(np.arange(0, d, 2, dtype=np.float32) * (-np.log(10000.0) / d))
    pe = np.zeros((max_len, d), dtype=np.float32)
    pe[:, 0::2] = np.sin(pos * div)
    pe[:, 1::2] = np.cos(pos * div)
    return pe


@functools.lru_cache(maxsize=None)
def _build(B, L, D, V):
    info = plsc.get_sparse_core_info()
    NC, NS = info.num_cores, info.num_subcores
    NW = NC * NS
    assert B % NW == 0
    n_chunks = B // NW
    assert n_chunks % _NBUF == 0 and n_chunks >= _NBUF
    G0 = 128
    G1 = L - G0
    mesh = plsc.VectorSubcoreMesh(core_axis_name="c", subcore_axis_name="s")

    @functools.partial(
        pl.kernel,
        out_type=jax.ShapeDtypeStruct((B, L, _LANE), jnp.float32),
        mesh=mesh,
        scratch_types=[pltpu.VMEM_SHARED((L, _LANE), jnp.float32)]
        + [pltpu.VMEM((L,), jnp.int32) for _ in range(_NBUF)]
        + [pltpu.VMEM((L, _LANE), jnp.float32) for _ in range(_NBUF)]
        + [pltpu.SemaphoreType.DMA for _ in range(4 * _NBUF)],
        compiler_params=pltpu.CompilerParams(use_tc_tiling_on_sc=True),
    )
    def emb(x_hbm, pe_hbm, table_hbm, out_hbm, pe_v, *refs):
        idxs = refs[:_NBUF]
        bufs = refs[_NBUF : 2 * _NBUF]
        sem_i = refs[2 * _NBUF : 3 * _NBUF]
        sem_g = refs[3 * _NBUF : 4 * _NBUF]
        sem_o = refs[4 * _NBUF : 5 * _NBUF]
        sem_p = refs[5 * _NBUF :]

        wid = lax.axis_index("s") * NC + lax.axis_index("c")
        seq0 = wid * n_chunks

        @pl.when(lax.axis_index("s") == 0)
        def _():
            pltpu.sync_copy(pe_hbm, pe_v)

        plsc.subcore_barrier()

        def start_idx(c, b):
            pltpu.async_copy(x_hbm.at[seq0 + c], idxs[b], sem_i[b])

        def wait_idx(b):
            pltpu.make_async_copy(x_hbm.at[0], idxs[b], sem_i[b]).wait()

        def start_prefill(b):
            pltpu.async_copy(pe_v, bufs[b], sem_p[b])

        def wait_prefill(b):
            pltpu.make_async_copy(pe_v, bufs[b], sem_p[b]).wait()

        def start_gather(b):
            pltpu.async_copy(
                table_hbm.at[idxs[b].at[pl.ds(0, G0)]],
                bufs[b].at[pl.ds(0, G0)],
                sem_g[b],
                add=True,
            )
            pltpu.async_copy(
                table_hbm.at[idxs[b].at[pl.ds(G0, G1)]],
                bufs[b].at[pl.ds(G0, G1)],
                sem_g[b],
                add=True,
            )

        def wait_gather(b):
            pltpu.make_async_copy(
                table_hbm.at[idxs[b].at[pl.ds(0, G0)]], bufs[b].at[pl.ds(0, G0)], sem_g[b]
            ).wait()
            pltpu.make_async_copy(
                table_hbm.at[idxs[b].at[pl.ds(0, G1)]], bufs[b].at[pl.ds(G0, G1)], sem_g[b]
            ).wait()

        def start_out(c, b):
            pltpu.async_copy(bufs[b], out_hbm.at[seq0 + c], sem_o[b])

        def wait_out(b):
            pltpu.make_async_copy(bufs[b], out_hbm.at[0], sem_o[b]).wait()

        for b in range(3):
            start_idx(b, b)
            start_prefill(b)
        for b in range(2):
            wait_idx(b)
            wait_prefill(b)
            start_gather(b)

        def grp_body(grp, carry):
            c_base = grp * _NBUF
            for bb in range(_NBUF):
                c = c_base + bb
                wait_gather(bb)
                start_out(c, bb)

                c3 = c + 3
                b3 = (bb + 3) % _NBUF

                @pl.when(jnp.logical_and(c3 < n_chunks, c3 >= _NBUF))
                def _():
                    wait_out(b3)

                @pl.when(c3 < n_chunks)
                def _():
                    start_idx(c3, b3)
                    start_prefill(b3)

                c1 = c + 2
                b1 = (bb + 2) % _NBUF

                @pl.when(c1 < n_chunks)
                def _():
                    wait_idx(b1)
                    wait_prefill(b1)
                    start_gather(b1)

            return carry

        lax.fori_loop(0, n_chunks // _NBUF, grp_body, 0)
        for b in range(_NBUF):
            wait_out(b)

    return emb


def kernel(x, table):
    B, L = x.shape
    V, D = table.shape
    pe = np.zeros((L, _LANE), dtype=np.float32)
    pe[:, :D] = _sin_pe(L, D)
    table_wide = jnp.pad(table, ((0, 0), (0, _LANE - D)))
    out_wide = _build(B, L, D, V)(x.astype(jnp.int32), jnp.asarray(pe), table_wide)
    return out_wide[:, :, :D]
